# speedup vs baseline: 2.2259x; 2.2259x over previous
import functools

import jax
import jax.numpy as jnp
from jax import lax
from jax.experimental import pallas as pl
from jax.experimental.pallas import tpu as pltpu

_NUM_BITS = 8
_THR = 4.0
_B, _F, _H, _K, _C = 1024, 128, 4096, 16, 10
_E = _F * _NUM_BITS

_HB = 512
_RB = 1024


def _encode_body(x_ref, code_ref):
    x = x_ref[...]
    levels = jnp.round(jnp.clip(x, 0.0, 1.0) * (2 ** _NUM_BITS - 1)).astype(jnp.int32)
    gray = levels ^ (levels >> 1)
    parts = [((gray >> j) & 1).astype(jnp.float32) for j in range(_NUM_BITS)]
    code_ref[...] = jnp.concatenate(parts, axis=1)


def _densify_body(idx_ref, w_ref, out_ref, *, permute_rows, row_block):
    r = pl.program_id(0)
    rv = lax.broadcasted_iota(jnp.int32, (row_block, _HB), 0) + r * row_block
    if permute_rows:
        rv = (rv % _F) * _NUM_BITS + rv // _F
    acc = jnp.zeros((row_block, _HB), jnp.float32)
    for k in range(_K):
        idxk = idx_ref[:, k][None, :]
        wk = w_ref[:, k][None, :]
        acc = acc + jnp.where(rv == idxk, wk, 0.0)
    out_ref[...] = acc


def _mm_thresh_body(a_ref, w_ref, out_ref):
    z = jnp.dot(a_ref[...], w_ref[...], preferred_element_type=jnp.float32)
    out_ref[...] = (z >= _THR).astype(jnp.float32)


def _out_body(a1_ref, a2_ref, oc0_ref, oc1_ref, out_ref):
    @pl.when(pl.program_id(0) == 0)
    def _():
        out_ref[...] = jnp.zeros_like(out_ref)

    out_ref[...] += (
        jnp.dot(a1_ref[...], oc0_ref[...], preferred_element_type=jnp.float32)
        + jnp.dot(a2_ref[...], oc1_ref[...], preferred_element_type=jnp.float32)
    )


def kernel(x, idx1, w1, idx2, w2, out_conn):
    code = pl.pallas_call(
        _encode_body,
        out_shape=jax.ShapeDtypeStruct((_B, _E), jnp.float32),
    )(x)

    dense1 = pl.pallas_call(
        functools.partial(_densify_body, permute_rows=True, row_block=_E),
        grid=(1, _H // _HB),
        in_specs=[
            pl.BlockSpec((_HB, _K), lambda r, c: (c, 0)),
            pl.BlockSpec((_HB, _K), lambda r, c: (c, 0)),
        ],
        out_specs=pl.BlockSpec((_E, _HB), lambda r, c: (r, c)),
        out_shape=jax.ShapeDtypeStruct((_E, _H), jnp.float32),
    )

    dense2 = pl.pallas_call(
        functools.partial(_densify_body, permute_rows=False, row_block=_RB),
        grid=(_H // _RB, _H // _HB),
        in_specs=[
            pl.BlockSpec((_HB, _K), lambda r, c: (c, 0)),
            pl.BlockSpec((_HB, _K), lambda r, c: (c, 0)),
        ],
        out_specs=pl.BlockSpec((_RB, _HB), lambda r, c: (r, c)),
        out_shape=jax.ShapeDtypeStruct((_H, _H), jnp.float32),
    )

    w1d = dense1(idx1, w1)
    w2d = dense2(idx2, w2)

    a1 = pl.pallas_call(
        _mm_thresh_body,
        grid=(_H // _HB,),
        in_specs=[
            pl.BlockSpec((_B, _E), lambda c: (0, 0)),
            pl.BlockSpec((_E, _HB), lambda c: (0, c)),
        ],
        out_specs=pl.BlockSpec((_B, _HB), lambda c: (0, c)),
        out_shape=jax.ShapeDtypeStruct((_B, _H), jnp.float32),
    )(code, w1d)

    a2 = pl.pallas_call(
        _mm_thresh_body,
        grid=(_H // _HB,),
        in_specs=[
            pl.BlockSpec((_B, _H), lambda c: (0, 0)),
            pl.BlockSpec((_H, _HB), lambda c: (0, c)),
        ],
        out_specs=pl.BlockSpec((_B, _HB), lambda c: (0, c)),
        out_shape=jax.ShapeDtypeStruct((_B, _H), jnp.float32),
    )(a1, w2d)

    out = pl.pallas_call(
        _out_body,
        grid=(_H // _HB,),
        in_specs=[
            pl.BlockSpec((_B, _HB), lambda c: (0, c)),
            pl.BlockSpec((_B, _HB), lambda c: (0, c)),
            pl.BlockSpec((_HB, _C), lambda c: (c, 0)),
            pl.BlockSpec((_HB, _C), lambda c: (c, 0)),
        ],
        out_specs=pl.BlockSpec((_B, _C), lambda c: (0, 0)),
        out_shape=jax.ShapeDtypeStruct((_B, _C), jnp.float32),
    )(a1, a2, out_conn[0], out_conn[1])

    return out


# speedup vs baseline: 3.9553x; 1.7770x over previous
import functools

import jax
import jax.numpy as jnp
from jax import lax
from jax.experimental import pallas as pl
from jax.experimental.pallas import tpu as pltpu
from jax.experimental.pallas import tpu_sc as plsc

_NUM_BITS = 8
_THR = 4.0
_B, _F, _H, _K, _C = 1024, 128, 4096, 16, 10
_E = _F * _NUM_BITS

_HB = 512
_NW = 32


def _encode_body(x_ref, code_ref):
    x = x_ref[...]
    levels = jnp.round(jnp.clip(x, 0.0, 1.0) * (2 ** _NUM_BITS - 1)).astype(jnp.int32)
    gray = levels ^ (levels >> 1)
    parts = [((gray >> j) & 1).astype(jnp.float32) for j in range(_NUM_BITS)]
    code_ref[...] = jnp.concatenate(parts, axis=1)


def _mm_thresh_nt_body(a_ref, m_ref, out_ref):
    z = lax.dot_general(
        a_ref[...], m_ref[...], (((1,), (1,)), ((), ())),
        preferred_element_type=jnp.float32,
    )
    out_ref[...] = (z >= _THR).astype(jnp.float32)


def _out_body(a1_ref, a2_ref, oc0_ref, oc1_ref, out_ref):
    @pl.when(pl.program_id(0) == 0)
    def _():
        out_ref[...] = jnp.zeros_like(out_ref)

    out_ref[...] += (
        jnp.dot(a1_ref[...], oc0_ref[...], preferred_element_type=jnp.float32)
        + jnp.dot(a2_ref[...], oc1_ref[...], preferred_element_type=jnp.float32)
    )


def _densify_sc_body(idx_hbm, w_hbm, out_hbm, buf, idx_v, w_v,
                     *, rows, chunk, permute):
    wid = lax.axis_index("s") * 2 + lax.axis_index("c")
    lane = lax.iota(jnp.int32, 16)

    def zero_body(i, _):
        buf[pl.ds(i * 16, 16)] = jnp.zeros((16,), jnp.float32)
        return 0
    lax.fori_loop(0, (chunk * rows) // 16, zero_body, 0, unroll=8)

    rows_per_worker = _H // _NW
    n_chunks = rows_per_worker // chunk
    groups = chunk // 16

    pltpu.sync_copy(idx_hbm.at[:, pl.ds(wid * rows_per_worker, rows_per_worker)], idx_v)
    pltpu.sync_copy(w_hbm.at[:, pl.ds(wid * rows_per_worker, rows_per_worker)], w_v)

    def scatter_all(c, sign):
        for k in range(_K):
            for g in range(groups):
                ev = idx_v[k, pl.ds(c * chunk + g * 16, 16)]
                if permute:
                    ev = (ev & 7) * 128 + (ev >> 3)
                wv = w_v[k, pl.ds(c * chunk + g * 16, 16)]
                addr = (lane + g * 16) * rows + ev
                plsc.addupdate_scatter(buf, [addr], sign * wv)

    for c in range(n_chunks):
        h0 = wid * rows_per_worker + c * chunk
        scatter_all(c, 1.0)
        pltpu.sync_copy(buf, out_hbm.at[pl.ds(h0 * rows, chunk * rows)])
        if c != n_chunks - 1:
            scatter_all(c, -1.0)


def _make_densify(rows, chunk, permute):
    mesh = plsc.VectorSubcoreMesh(core_axis_name="c", subcore_axis_name="s")
    return pl.kernel(
        functools.partial(_densify_sc_body, rows=rows, chunk=chunk,
                          permute=permute),
        mesh=mesh,
        compiler_params=pltpu.CompilerParams(
            use_tc_tiling_on_sc=False, needs_layout_passes=False),
        out_type=jax.ShapeDtypeStruct((_H * rows,), jnp.float32),
        scratch_types=[
            pltpu.VMEM((chunk * rows,), jnp.float32),
            pltpu.VMEM((_K, _H // _NW), jnp.int32),
            pltpu.VMEM((_K, _H // _NW), jnp.float32),
        ],
    )


def kernel(x, idx1, w1, idx2, w2, out_conn):
    code = pl.pallas_call(
        _encode_body,
        out_shape=jax.ShapeDtypeStruct((_B, _E), jnp.float32),
    )(x)

    m1 = _make_densify(_E, 32, True)(idx1.T, w1.T)
    m1 = m1.reshape(_H, _E)
    m2 = _make_densify(_H, 16, False)(idx2.T, w2.T)
    m2 = m2.reshape(_H, _H)

    a1 = pl.pallas_call(
        _mm_thresh_nt_body,
        grid=(_H // _HB,),
        in_specs=[
            pl.BlockSpec((_B, _E), lambda c: (0, 0)),
            pl.BlockSpec((_HB, _E), lambda c: (c, 0)),
        ],
        out_specs=pl.BlockSpec((_B, _HB), lambda c: (0, c)),
        out_shape=jax.ShapeDtypeStruct((_B, _H), jnp.float32),
    )(code, m1)

    a2 = pl.pallas_call(
        _mm_thresh_nt_body,
        grid=(_H // _HB,),
        in_specs=[
            pl.BlockSpec((_B, _H), lambda c: (0, 0)),
            pl.BlockSpec((_HB, _H), lambda c: (c, 0)),
        ],
        out_specs=pl.BlockSpec((_B, _HB), lambda c: (0, c)),
        out_shape=jax.ShapeDtypeStruct((_B, _H), jnp.float32),
    )(a1, m2)

    out = pl.pallas_call(
        _out_body,
        grid=(_H // _HB,),
        in_specs=[
            pl.BlockSpec((_B, _HB), lambda c: (0, c)),
            pl.BlockSpec((_B, _HB), lambda c: (0, c)),
            pl.BlockSpec((_HB, _C), lambda c: (c, 0)),
            pl.BlockSpec((_HB, _C), lambda c: (c, 0)),
        ],
        out_specs=pl.BlockSpec((_B, _C), lambda c: (0, 0)),
        out_shape=jax.ShapeDtypeStruct((_B, _C), jnp.float32),
    )(a1, a2, out_conn[0], out_conn[1])

    return out


# speedup vs baseline: 4.0861x; 1.0331x over previous
import functools

import jax
import jax.numpy as jnp
from jax import lax
from jax.experimental import pallas as pl
from jax.experimental.pallas import tpu as pltpu
from jax.experimental.pallas import tpu_sc as plsc

_NUM_BITS = 8
_THR = 4.0
_B, _F, _H, _K, _C = 1024, 128, 4096, 16, 10
_E = _F * _NUM_BITS

_HB = 512
_NW = 32


def _encode_body(x_ref, code_ref):
    x = x_ref[...]
    levels = jnp.round(jnp.clip(x, 0.0, 1.0) * (2 ** _NUM_BITS - 1)).astype(jnp.int32)
    gray = levels ^ (levels >> 1)
    parts = [((gray >> j) & 1).astype(jnp.bfloat16) for j in range(_NUM_BITS)]
    code_ref[...] = jnp.concatenate(parts, axis=1)


def _mm_thresh_nt_body(a_ref, m_ref, out_ref):
    m = m_ref[...].astype(jnp.bfloat16)
    z = lax.dot_general(
        a_ref[...], m, (((1,), (1,)), ((), ())),
        preferred_element_type=jnp.float32,
    )
    out_ref[...] = (z >= _THR).astype(jnp.bfloat16)


def _out_body(a1_ref, a2_ref, oc0_ref, oc1_ref, out_ref):
    @pl.when(pl.program_id(0) == 0)
    def _():
        out_ref[...] = jnp.zeros_like(out_ref)

    a1 = a1_ref[...].astype(jnp.float32)
    a2 = a2_ref[...].astype(jnp.float32)
    out_ref[...] += (
        jnp.dot(a1, oc0_ref[...], preferred_element_type=jnp.float32)
        + jnp.dot(a2, oc1_ref[...], preferred_element_type=jnp.float32)
    )


def _densify_sc_body(idx_hbm, w_hbm, out_hbm, buf, idx_v, w_v,
                     *, rows, chunk, permute):
    wid = lax.axis_index("s") * 2 + lax.axis_index("c")
    lane = lax.iota(jnp.int32, 16)

    def zero_body(i, _):
        buf[pl.ds(i * 16, 16)] = jnp.zeros((16,), jnp.float32)
        return 0
    lax.fori_loop(0, (chunk * rows) // 16, zero_body, 0, unroll=8)

    rows_per_worker = _H // _NW
    n_chunks = rows_per_worker // chunk
    groups = chunk // 16

    pltpu.sync_copy(idx_hbm.at[:, pl.ds(wid * rows_per_worker, rows_per_worker)], idx_v)
    pltpu.sync_copy(w_hbm.at[:, pl.ds(wid * rows_per_worker, rows_per_worker)], w_v)

    def scatter_all(c, sign):
        for k in range(_K):
            for g in range(groups):
                ev = idx_v[k, pl.ds(c * chunk + g * 16, 16)]
                if permute:
                    ev = (ev & 7) * 128 + (ev >> 3)
                wv = w_v[k, pl.ds(c * chunk + g * 16, 16)]
                addr = (lane + g * 16) * rows + ev
                plsc.addupdate_scatter(buf, [addr], sign * wv)

    for c in range(n_chunks):
        h0 = wid * rows_per_worker + c * chunk
        scatter_all(c, 1.0)
        pltpu.sync_copy(buf, out_hbm.at[pl.ds(h0 * rows, chunk * rows)])
        if c != n_chunks - 1:
            scatter_all(c, -1.0)


def _make_densify(rows, chunk, permute):
    mesh = plsc.VectorSubcoreMesh(core_axis_name="c", subcore_axis_name="s")
    return pl.kernel(
        functools.partial(_densify_sc_body, rows=rows, chunk=chunk,
                          permute=permute),
        mesh=mesh,
        compiler_params=pltpu.CompilerParams(
            use_tc_tiling_on_sc=False, needs_layout_passes=False),
        out_type=jax.ShapeDtypeStruct((_H * rows,), jnp.float32),
        scratch_types=[
            pltpu.VMEM((chunk * rows,), jnp.float32),
            pltpu.VMEM((_K, _H // _NW), jnp.int32),
            pltpu.VMEM((_K, _H // _NW), jnp.float32),
        ],
    )


def kernel(x, idx1, w1, idx2, w2, out_conn):
    code = pl.pallas_call(
        _encode_body,
        out_shape=jax.ShapeDtypeStruct((_B, _E), jnp.bfloat16),
    )(x)

    m1 = _make_densify(_E, 32, True)(idx1.T, w1.T)
    m1 = m1.reshape(_H, _E)
    m2 = _make_densify(_H, 16, False)(idx2.T, w2.T)
    m2 = m2.reshape(_H, _H)

    a1 = pl.pallas_call(
        _mm_thresh_nt_body,
        grid=(_H // _HB,),
        in_specs=[
            pl.BlockSpec((_B, _E), lambda c: (0, 0)),
            pl.BlockSpec((_HB, _E), lambda c: (c, 0)),
        ],
        out_specs=pl.BlockSpec((_B, _HB), lambda c: (0, c)),
        out_shape=jax.ShapeDtypeStruct((_B, _H), jnp.bfloat16),
    )(code, m1)

    a2 = pl.pallas_call(
        _mm_thresh_nt_body,
        grid=(_H // _HB,),
        in_specs=[
            pl.BlockSpec((_B, _H), lambda c: (0, 0)),
            pl.BlockSpec((_HB, _H), lambda c: (c, 0)),
        ],
        out_specs=pl.BlockSpec((_B, _HB), lambda c: (0, c)),
        out_shape=jax.ShapeDtypeStruct((_B, _H), jnp.bfloat16),
    )(a1, m2)

    out = pl.pallas_call(
        _out_body,
        grid=(_H // _HB,),
        in_specs=[
            pl.BlockSpec((_B, _HB), lambda c: (0, c)),
            pl.BlockSpec((_B, _HB), lambda c: (0, c)),
            pl.BlockSpec((_HB, _C), lambda c: (c, 0)),
            pl.BlockSpec((_HB, _C), lambda c: (c, 0)),
        ],
        out_specs=pl.BlockSpec((_B, _C), lambda c: (0, 0)),
        out_shape=jax.ShapeDtypeStruct((_B, _C), jnp.float32),
    )(a1, a2, out_conn[0], out_conn[1])

    return out


# speedup vs baseline: 6.5682x; 1.6075x over previous
import functools

import jax
import jax.numpy as jnp
from jax import lax
from jax.experimental import pallas as pl
from jax.experimental.pallas import tpu as pltpu
from jax.experimental.pallas import tpu_sc as plsc

_NUM_BITS = 8
_THR = 4.0
_B, _F, _H, _K, _C = 1024, 128, 4096, 16, 10
_E = _F * _NUM_BITS

_HB = 512
_NW = 32


def _encode_body(x_ref, code_ref):
    x = x_ref[...]
    levels = jnp.round(jnp.clip(x, 0.0, 1.0) * (2 ** _NUM_BITS - 1)).astype(jnp.int32)
    gray = levels ^ (levels >> 1)
    parts = [((gray >> j) & 1).astype(jnp.bfloat16) for j in range(_NUM_BITS)]
    code_ref[...] = jnp.concatenate(parts, axis=1)


def _mm_thresh_nt_body(a_ref, m_ref, out_ref):
    m = m_ref[...].astype(jnp.bfloat16)
    z = lax.dot_general(
        a_ref[...], m, (((1,), (1,)), ((), ())),
        preferred_element_type=jnp.float32,
    )
    out_ref[...] = (z >= _THR).astype(jnp.bfloat16)


def _out_body(a1_ref, a2_ref, oc0_ref, oc1_ref, out_ref):
    @pl.when(pl.program_id(0) == 0)
    def _():
        out_ref[...] = jnp.zeros_like(out_ref)

    a1 = a1_ref[...].astype(jnp.float32)
    a2 = a2_ref[...].astype(jnp.float32)
    out_ref[...] += (
        jnp.dot(a1, oc0_ref[...], preferred_element_type=jnp.float32)
        + jnp.dot(a2, oc1_ref[...], preferred_element_type=jnp.float32)
    )


def _densify_sc_body(idx_hbm, w_hbm, out_hbm, buf, idx_v, w_v,
                     *, rows, chunk, permute):
    wid = lax.axis_index("s") * 2 + lax.axis_index("c")
    lane = lax.iota(jnp.int32, 16)

    for r in range(chunk):
        def zero_body(j, _, r=r):
            buf[r, pl.ds(j * 16, 16)] = jnp.zeros((16,), jnp.float32)
            return 0
        lax.fori_loop(0, rows // 16, zero_body, 0, unroll=8)

    rows_per_worker = _H // _NW
    n_chunks = rows_per_worker // chunk
    groups = chunk // 16

    pltpu.sync_copy(idx_hbm.at[:, pl.ds(wid * rows_per_worker, rows_per_worker)], idx_v)
    pltpu.sync_copy(w_hbm.at[:, pl.ds(wid * rows_per_worker, rows_per_worker)], w_v)

    def scatter_all(c, sign):
        for k in range(_K):
            for g in range(groups):
                ev = idx_v[k, pl.ds(c * chunk + g * 16, 16)]
                if permute:
                    ev = (ev & 7) * 128 + (ev >> 3)
                wv = w_v[k, pl.ds(c * chunk + g * 16, 16)]
                plsc.addupdate_scatter(buf, [lane + g * 16, ev], sign * wv)

    for c in range(n_chunks):
        h0 = wid * rows_per_worker + c * chunk
        scatter_all(c, 1.0)
        pltpu.sync_copy(buf, out_hbm.at[pl.ds(h0, chunk), :])
        if c != n_chunks - 1:
            scatter_all(c, -1.0)


def _make_densify(rows, chunk, permute):
    mesh = plsc.VectorSubcoreMesh(core_axis_name="c", subcore_axis_name="s")
    return pl.kernel(
        functools.partial(_densify_sc_body, rows=rows, chunk=chunk,
                          permute=permute),
        mesh=mesh,
        compiler_params=pltpu.CompilerParams(needs_layout_passes=False),
        out_type=jax.ShapeDtypeStruct((_H, rows), jnp.float32),
        scratch_types=[
            pltpu.VMEM((chunk, rows), jnp.float32),
            pltpu.VMEM((_K, _H // _NW), jnp.int32),
            pltpu.VMEM((_K, _H // _NW), jnp.float32),
        ],
    )


def kernel(x, idx1, w1, idx2, w2, out_conn):
    code = pl.pallas_call(
        _encode_body,
        out_shape=jax.ShapeDtypeStruct((_B, _E), jnp.bfloat16),
    )(x)

    m1 = _make_densify(_E, 32, True)(idx1.T, w1.T)
    m2 = _make_densify(_H, 16, False)(idx2.T, w2.T)

    a1 = pl.pallas_call(
        _mm_thresh_nt_body,
        grid=(_H // _HB,),
        in_specs=[
            pl.BlockSpec((_B, _E), lambda c: (0, 0)),
            pl.BlockSpec((_HB, _E), lambda c: (c, 0)),
        ],
        out_specs=pl.BlockSpec((_B, _HB), lambda c: (0, c)),
        out_shape=jax.ShapeDtypeStruct((_B, _H), jnp.bfloat16),
    )(code, m1)

    a2 = pl.pallas_call(
        _mm_thresh_nt_body,
        grid=(_H // _HB,),
        in_specs=[
            pl.BlockSpec((_B, _H), lambda c: (0, 0)),
            pl.BlockSpec((_HB, _H), lambda c: (c, 0)),
        ],
        out_specs=pl.BlockSpec((_B, _HB), lambda c: (0, c)),
        out_shape=jax.ShapeDtypeStruct((_B, _H), jnp.bfloat16),
    )(a1, m2)

    out = pl.pallas_call(
        _out_body,
        grid=(_H // _HB,),
        in_specs=[
            pl.BlockSpec((_B, _HB), lambda c: (0, c)),
            pl.BlockSpec((_B, _HB), lambda c: (0, c)),
            pl.BlockSpec((_HB, _C), lambda c: (c, 0)),
            pl.BlockSpec((_HB, _C), lambda c: (c, 0)),
        ],
        out_specs=pl.BlockSpec((_B, _C), lambda c: (0, 0)),
        out_shape=jax.ShapeDtypeStruct((_B, _C), jnp.float32),
    )(a1, a2, out_conn[0], out_conn[1])

    return out
